# baseline (speedup 1.0000x reference)
import math

import jax
import jax.numpy as jnp
import numpy as np
from jax.experimental import pallas as pl
from jax.experimental.pallas import tpu as pltpu

_LOG_NC = math.log(256.0)
_NEG = float(np.log(np.float32(1e-30)))


def _qsample_kernel(t_ref, lca_ref, l1m_ref, lx_ref, u_ref, out_ref):
    b = pl.program_id(0)
    ti = t_ref[b]
    a = lca_ref[ti]
    c = l1m_ref[ti] - _LOG_NC

    lx = lx_ref[0]
    u = u_ref[0]
    gumbel = -jnp.log(-jnp.log(u + 1e-30) + 1e-30)
    xa = lx + a
    m = jnp.maximum(xa, c)
    log_probs = m + jnp.log(jnp.exp(xa - m) + jnp.exp(c - m))
    v = gumbel + log_probs

    idx = jnp.argmax(v, axis=0)
    cls = jax.lax.broadcasted_iota(jnp.int32, v.shape, 0)
    out_ref[0] = jnp.where(cls == idx[None, :], jnp.float32(0.0),
                           jnp.float32(_NEG))


def kernel(log_x_start, t, uniform, log_cumprod_alpha, log_1_min_cumprod_alpha):
    B, C, H, W = log_x_start.shape
    Hb = 16
    grid = (B, H // Hb)
    blk = pl.BlockSpec((1, C, Hb, W), lambda b, j: (b, 0, j, 0))
    out = pl.pallas_call(
        _qsample_kernel,
        grid=grid,
        in_specs=[
            pl.BlockSpec(memory_space=pltpu.SMEM),
            pl.BlockSpec(memory_space=pltpu.SMEM),
            pl.BlockSpec(memory_space=pltpu.SMEM),
            blk,
            blk,
        ],
        out_specs=blk,
        out_shape=jax.ShapeDtypeStruct((B, C, H, W), jnp.float32),
        compiler_params=pltpu.CompilerParams(
            dimension_semantics=("parallel", "parallel")),
    )(t, log_cumprod_alpha, log_1_min_cumprod_alpha, log_x_start, uniform)
    return out


# speedup vs baseline: 4.6423x; 4.6423x over previous
import math

import jax
import jax.numpy as jnp
import numpy as np
from jax.experimental import pallas as pl
from jax.experimental.pallas import tpu as pltpu

_LOG_NC = math.log(256.0)
_NEG = float(np.log(np.float32(1e-30)))


def _qsample_kernel(t_ref, lca_ref, l1m_ref, lx_ref, u_ref, out_ref):
    b = pl.program_id(0)
    ti = t_ref[b]
    a = lca_ref[ti]
    c = l1m_ref[ti] - _LOG_NC

    lx = lx_ref[0]
    u = u_ref[0]
    gumbel = -jnp.log(-jnp.log(u + 1e-30) + 1e-30)
    xa = lx + a
    m = jnp.maximum(xa, c)
    log_probs = m + jnp.log(jnp.exp(xa - m) + jnp.exp(c - m))
    v = gumbel + log_probs

    idx = jnp.argmax(v, axis=1)
    cls = jax.lax.broadcasted_iota(jnp.int32, v.shape, 1)
    out_ref[0] = jnp.where(cls == idx[:, None], jnp.float32(0.0),
                           jnp.float32(_NEG))


def kernel(log_x_start, t, uniform, log_cumprod_alpha, log_1_min_cumprod_alpha):
    B, C, H, W = log_x_start.shape
    HW = H * W
    S = 1024
    lx = jnp.transpose(log_x_start, (0, 2, 3, 1)).reshape(B, HW, C)
    u = jnp.transpose(uniform, (0, 2, 3, 1)).reshape(B, HW, C)
    grid = (B, HW // S)
    blk = pl.BlockSpec((1, S, C), lambda b, j: (b, j, 0))
    out = pl.pallas_call(
        _qsample_kernel,
        grid=grid,
        in_specs=[
            pl.BlockSpec(memory_space=pltpu.SMEM),
            pl.BlockSpec(memory_space=pltpu.SMEM),
            pl.BlockSpec(memory_space=pltpu.SMEM),
            blk,
            blk,
        ],
        out_specs=blk,
        out_shape=jax.ShapeDtypeStruct((B, HW, C), jnp.float32),
        compiler_params=pltpu.CompilerParams(
            dimension_semantics=("parallel", "parallel")),
    )(t, log_cumprod_alpha, log_1_min_cumprod_alpha, lx, u)
    return jnp.transpose(out.reshape(B, H, W, C), (0, 3, 1, 2))


# speedup vs baseline: 5.0937x; 1.0972x over previous
import math

import jax
import jax.numpy as jnp
import numpy as np
from jax.experimental import pallas as pl
from jax.experimental.pallas import tpu as pltpu

_LOG_NC = math.log(256.0)
_NEG = float(np.log(np.float32(1e-30)))


def _qsample_kernel(t_ref, lca_ref, l1m_ref, lx_ref, u_ref, out_ref):
    b = pl.program_id(0)
    ti = t_ref[b]
    a = lca_ref[ti]
    k = jnp.exp(l1m_ref[ti] - _LOG_NC)

    lx = lx_ref[0]
    u = u_ref[0]
    e = -jnp.log(u + 1e-30) + 1e-30
    p = jnp.exp(lx + a) + k
    r = e / p

    idx = jnp.argmin(r, axis=1)
    cls = jax.lax.broadcasted_iota(jnp.int32, r.shape, 1)
    out_ref[0] = jnp.where(cls == idx[:, None], jnp.float32(0.0),
                           jnp.float32(_NEG))


def kernel(log_x_start, t, uniform, log_cumprod_alpha, log_1_min_cumprod_alpha):
    B, C, H, W = log_x_start.shape
    HW = H * W
    S = 1024
    lx = jnp.transpose(log_x_start, (0, 2, 3, 1)).reshape(B, HW, C)
    u = jnp.transpose(uniform, (0, 2, 3, 1)).reshape(B, HW, C)
    grid = (B, HW // S)
    blk = pl.BlockSpec((1, S, C), lambda b, j: (b, j, 0))
    out = pl.pallas_call(
        _qsample_kernel,
        grid=grid,
        in_specs=[
            pl.BlockSpec(memory_space=pltpu.SMEM),
            pl.BlockSpec(memory_space=pltpu.SMEM),
            pl.BlockSpec(memory_space=pltpu.SMEM),
            blk,
            blk,
        ],
        out_specs=blk,
        out_shape=jax.ShapeDtypeStruct((B, HW, C), jnp.float32),
        compiler_params=pltpu.CompilerParams(
            dimension_semantics=("parallel", "parallel")),
    )(t, log_cumprod_alpha, log_1_min_cumprod_alpha, lx, u)
    return jnp.transpose(out.reshape(B, H, W, C), (0, 3, 1, 2))


# speedup vs baseline: 6.4810x; 1.2724x over previous
import math

import jax
import jax.numpy as jnp
import numpy as np
from jax.experimental import pallas as pl
from jax.experimental.pallas import tpu as pltpu

_LOG_NC = math.log(256.0)
_NEG = float(np.log(np.float32(1e-30)))


def _qsample_kernel(t_ref, lca_ref, l1m_ref, lx_ref, u_ref, out_ref):
    b = pl.program_id(0)
    ti = t_ref[b]
    a = lca_ref[ti]
    k = jnp.exp(l1m_ref[ti] - _LOG_NC)

    lx = lx_ref[0]
    u = u_ref[0]
    e = -jnp.log(u + 1e-30) + 1e-30
    p = jnp.exp(lx + a) + k
    r = e / p

    idx = jnp.argmin(r, axis=1)
    cls = jax.lax.broadcasted_iota(jnp.int32, r.shape, 1)
    out_ref[0] = jnp.where(cls == idx[:, None], jnp.float32(0.0),
                           jnp.float32(_NEG))


def kernel(log_x_start, t, uniform, log_cumprod_alpha, log_1_min_cumprod_alpha):
    B, C, H, W = log_x_start.shape
    HW = H * W
    S = 2048
    lx = jnp.transpose(log_x_start, (0, 2, 3, 1)).reshape(B, HW, C)
    u = jnp.transpose(uniform, (0, 2, 3, 1)).reshape(B, HW, C)
    grid = (B, HW // S)
    blk = pl.BlockSpec((1, S, C), lambda b, j: (b, j, 0))
    out = pl.pallas_call(
        _qsample_kernel,
        grid=grid,
        in_specs=[
            pl.BlockSpec(memory_space=pltpu.SMEM),
            pl.BlockSpec(memory_space=pltpu.SMEM),
            pl.BlockSpec(memory_space=pltpu.SMEM),
            blk,
            blk,
        ],
        out_specs=blk,
        out_shape=jax.ShapeDtypeStruct((B, HW, C), jnp.float32),
        compiler_params=pltpu.CompilerParams(
            dimension_semantics=("parallel", "parallel")),
    )(t, log_cumprod_alpha, log_1_min_cumprod_alpha, lx, u)
    return jnp.transpose(out.reshape(B, H, W, C), (0, 3, 1, 2))


# speedup vs baseline: 7.2095x; 1.1124x over previous
import math

import jax
import jax.numpy as jnp
import numpy as np
from jax.experimental import pallas as pl
from jax.experimental.pallas import tpu as pltpu

_LOG_NC = math.log(256.0)
_NEG = float(np.log(np.float32(1e-30)))


def _qsample_kernel(t_ref, lca_ref, l1m_ref, lx_ref, u_ref, out_ref):
    b = pl.program_id(0)
    ti = t_ref[b]
    a = lca_ref[ti]
    k = jnp.exp(l1m_ref[ti] - _LOG_NC)

    lx = lx_ref[0]
    u = u_ref[0]
    e = -jnp.log(u + 1e-30) + 1e-30
    p = jnp.exp(lx + a) + k
    r = e / p

    idx = jnp.argmin(r, axis=1)
    cls = jax.lax.broadcasted_iota(jnp.int32, r.shape, 1)
    out_ref[0] = jnp.where(cls == idx[:, None], jnp.float32(0.0),
                           jnp.float32(_NEG))


def kernel(log_x_start, t, uniform, log_cumprod_alpha, log_1_min_cumprod_alpha):
    B, C, H, W = log_x_start.shape
    HW = H * W
    S = 4096
    lx = jnp.transpose(log_x_start, (0, 2, 3, 1)).reshape(B, HW, C)
    u = jnp.transpose(uniform, (0, 2, 3, 1)).reshape(B, HW, C)
    grid = (B, HW // S)
    blk = pl.BlockSpec((1, S, C), lambda b, j: (b, j, 0))
    out = pl.pallas_call(
        _qsample_kernel,
        grid=grid,
        in_specs=[
            pl.BlockSpec(memory_space=pltpu.SMEM),
            pl.BlockSpec(memory_space=pltpu.SMEM),
            pl.BlockSpec(memory_space=pltpu.SMEM),
            blk,
            blk,
        ],
        out_specs=blk,
        out_shape=jax.ShapeDtypeStruct((B, HW, C), jnp.float32),
        compiler_params=pltpu.CompilerParams(
            dimension_semantics=("parallel", "parallel")),
    )(t, log_cumprod_alpha, log_1_min_cumprod_alpha, lx, u)
    return jnp.transpose(out.reshape(B, H, W, C), (0, 3, 1, 2))


# speedup vs baseline: 7.3324x; 1.0170x over previous
import math

import jax
import jax.numpy as jnp
import numpy as np
from jax.experimental import pallas as pl
from jax.experimental.pallas import tpu as pltpu

_LOG_NC = math.log(256.0)
_NEG = float(np.log(np.float32(1e-30)))


def _qsample_kernel(t_ref, lca_ref, l1m_ref, lx_ref, u_ref, out_ref):
    nb = lx_ref.shape[0]
    b0 = pl.program_id(0) * nb
    for i in range(nb):
        ti = t_ref[b0 + i]
        a = lca_ref[ti]
        k = jnp.exp(l1m_ref[ti] - _LOG_NC)

        lx = lx_ref[i]
        u = u_ref[i]
        e = -jnp.log(u + 1e-30) + 1e-30
        p = jnp.exp(lx + a) + k
        r = e / p

        idx = jnp.argmin(r, axis=1)
        cls = jax.lax.broadcasted_iota(jnp.int32, r.shape, 1)
        out_ref[i] = jnp.where(cls == idx[:, None], jnp.float32(0.0),
                               jnp.float32(_NEG))


def kernel(log_x_start, t, uniform, log_cumprod_alpha, log_1_min_cumprod_alpha):
    B, C, H, W = log_x_start.shape
    HW = H * W
    S = 4096
    NB = 2
    lx = jnp.transpose(log_x_start, (0, 2, 3, 1)).reshape(B, HW, C)
    u = jnp.transpose(uniform, (0, 2, 3, 1)).reshape(B, HW, C)
    grid = (B // NB, HW // S)
    blk = pl.BlockSpec((NB, S, C), lambda b, j: (b, j, 0))
    out = pl.pallas_call(
        _qsample_kernel,
        grid=grid,
        in_specs=[
            pl.BlockSpec(memory_space=pltpu.SMEM),
            pl.BlockSpec(memory_space=pltpu.SMEM),
            pl.BlockSpec(memory_space=pltpu.SMEM),
            blk,
            blk,
        ],
        out_specs=blk,
        out_shape=jax.ShapeDtypeStruct((B, HW, C), jnp.float32),
        compiler_params=pltpu.CompilerParams(
            dimension_semantics=("parallel", "parallel")),
    )(t, log_cumprod_alpha, log_1_min_cumprod_alpha, lx, u)
    return jnp.transpose(out.reshape(B, H, W, C), (0, 3, 1, 2))
